# baseline (speedup 1.0000x reference)
import functools

import jax
import jax.numpy as jnp
from jax import lax
from jax.experimental import pallas as pl
from jax.experimental.pallas import tpu as pltpu


def _round_up(x, m):
    return ((x + m - 1) // m) * m


def _fused_kernel(feat_ref, alpha_ref, wqT_ref, bq_ref, w1T_ref, w2T_ref,
                  bw_ref, out_ref, h_ref, *, nk, tm):
    i0 = pl.program_id(0)
    k = pl.program_id(1)

    @pl.when(k == 0)
    def _():
        x = feat_ref[...].astype(jnp.bfloat16)
        acc = jnp.dot(x, wqT_ref[...], preferred_element_type=jnp.float32)
        h_ref[...] = jnp.maximum(acc + bq_ref[...], 0.0).astype(jnp.bfloat16)

    a16 = alpha_ref[...].astype(jnp.bfloat16)
    hn = jnp.dot(a16, h_ref[...], preferred_element_type=jnp.float32)

    row0 = (i0 * nk + k) * tm
    hd = h_ref[pl.ds(row0, tm), :]
    z = jnp.dot(hd, w1T_ref[...], preferred_element_type=jnp.float32)
    z = z + jnp.dot(hn.astype(jnp.bfloat16), w2T_ref[...],
                    preferred_element_type=jnp.float32)
    z = jnp.maximum(z + bw_ref[...], 0.0)
    sumsq = jnp.sum(z * z, axis=-1, keepdims=True)
    inv_norm = lax.rsqrt(sumsq + 1e-12)
    out_ref[...] = (z * inv_norm).astype(out_ref.dtype)


def kernel(features, alpha, wq, bq, ww, bw):
    n, in_dim = features.shape
    out_dim = ww.shape[0]
    dtype = features.dtype

    d_pad = _round_up(in_dim, 128)
    o_pad = _round_up(out_dim, 128)
    n_pad = _round_up(n, 128)

    def pad2(x, r, c):
        if x.shape == (r, c):
            return x
        return jnp.pad(x, ((0, r - x.shape[0]), (0, c - x.shape[1])))

    feat_p = pad2(features, n_pad, d_pad)
    alpha_p = pad2(alpha, n_pad, n_pad)
    wqT_p = pad2(wq.T, d_pad, d_pad).astype(jnp.bfloat16)
    bq_p = pad2(bq.reshape(1, in_dim), 1, d_pad)
    w1T_p = pad2(ww[:, :in_dim].T, d_pad, o_pad).astype(jnp.bfloat16)
    w2T_p = pad2(ww[:, in_dim:].T, d_pad, o_pad).astype(jnp.bfloat16)
    bw_p = pad2(bw.reshape(1, out_dim), 1, o_pad)

    tm = 256 if n_pad % 512 == 0 else 128
    nk = n_pad // (2 * tm)

    out_p = pl.pallas_call(
        functools.partial(_fused_kernel, nk=nk, tm=tm),
        out_shape=jax.ShapeDtypeStruct((n_pad, o_pad), dtype),
        grid=(2, nk),
        in_specs=[
            pl.BlockSpec((n_pad, d_pad), lambda i, k: (0, 0)),
            pl.BlockSpec((tm, n_pad), lambda i, k, nk=nk: (i * nk + k, 0)),
            pl.BlockSpec((d_pad, d_pad), lambda i, k: (0, 0)),
            pl.BlockSpec((1, d_pad), lambda i, k: (0, 0)),
            pl.BlockSpec((d_pad, o_pad), lambda i, k: (0, 0)),
            pl.BlockSpec((d_pad, o_pad), lambda i, k: (0, 0)),
            pl.BlockSpec((1, o_pad), lambda i, k: (0, 0)),
        ],
        out_specs=pl.BlockSpec((tm, o_pad), lambda i, k: (i * nk + k, 0)),
        scratch_shapes=[pltpu.VMEM((n_pad, d_pad), jnp.bfloat16)],
        compiler_params=pltpu.CompilerParams(
            dimension_semantics=("parallel", "arbitrary"),
            vmem_limit_bytes=64 * 1024 * 1024),
    )(feat_p, alpha_p, wqT_p, bq_p, w1T_p, w2T_p, bw_p)

    return out_p[:n, :out_dim]


# speedup vs baseline: 1.1085x; 1.1085x over previous
import functools

import jax
import jax.numpy as jnp
from jax import lax
from jax.experimental import pallas as pl
from jax.experimental.pallas import tpu as pltpu


def _round_up(x, m):
    return ((x + m - 1) // m) * m


def _fused_kernel(feat_ref, alphaL_ref, alphaR_ref, wqT_ref, bq_ref, w1T_ref,
                  w2T_ref, bw_ref, out_ref, h_ref, *, nk, tm, nh):
    i0 = pl.program_id(0)
    k = pl.program_id(1)

    @pl.when(k == 0)
    def _():
        x = feat_ref[...].astype(jnp.bfloat16)
        acc = jnp.dot(x, wqT_ref[...], preferred_element_type=jnp.float32)
        h_ref[...] = jnp.maximum(acc + bq_ref[...], 0.0).astype(jnp.bfloat16)

    hn = jnp.dot(alphaL_ref[...].astype(jnp.bfloat16), h_ref[:nh, :],
                 preferred_element_type=jnp.float32)
    hn = hn + jnp.dot(alphaR_ref[...].astype(jnp.bfloat16), h_ref[nh:, :],
                      preferred_element_type=jnp.float32)

    row0 = (i0 * nk + k) * tm
    hd = h_ref[pl.ds(row0, tm), :]
    z = jnp.dot(hd, w1T_ref[...], preferred_element_type=jnp.float32)
    z = z + jnp.dot(hn.astype(jnp.bfloat16), w2T_ref[...],
                    preferred_element_type=jnp.float32)
    z = jnp.maximum(z + bw_ref[...], 0.0)
    sumsq = jnp.sum(z * z, axis=-1, keepdims=True)
    inv_norm = lax.rsqrt(sumsq + 1e-12)
    out_ref[...] = (z * inv_norm).astype(out_ref.dtype)


def kernel(features, alpha, wq, bq, ww, bw):
    n, in_dim = features.shape
    out_dim = ww.shape[0]
    dtype = features.dtype

    d_pad = _round_up(in_dim, 128)
    o_pad = _round_up(out_dim, 128)
    n_pad = _round_up(n, 128)

    def pad2(x, r, c):
        if x.shape == (r, c):
            return x
        return jnp.pad(x, ((0, r - x.shape[0]), (0, c - x.shape[1])))

    feat_p = pad2(features, n_pad, d_pad)
    alpha_p = pad2(alpha, n_pad, n_pad)
    wqT_p = pad2(wq.T, d_pad, d_pad).astype(jnp.bfloat16)
    bq_p = pad2(bq.reshape(1, in_dim), 1, d_pad)
    w1T_p = pad2(ww[:, :in_dim].T, d_pad, o_pad).astype(jnp.bfloat16)
    w2T_p = pad2(ww[:, in_dim:].T, d_pad, o_pad).astype(jnp.bfloat16)
    bw_p = pad2(bw.reshape(1, out_dim), 1, o_pad)

    tm = 512 if n_pad % 1024 == 0 else 128
    nk = n_pad // (2 * tm)
    nh = n_pad // 2

    out_p = pl.pallas_call(
        functools.partial(_fused_kernel, nk=nk, tm=tm, nh=nh),
        out_shape=jax.ShapeDtypeStruct((n_pad, o_pad), dtype),
        grid=(2, nk),
        in_specs=[
            pl.BlockSpec((n_pad, d_pad), lambda i, k: (0, 0)),
            pl.BlockSpec((tm, nh), lambda i, k, nk=nk: (i * nk + k, 0)),
            pl.BlockSpec((tm, nh), lambda i, k, nk=nk: (i * nk + k, 1)),
            pl.BlockSpec((d_pad, d_pad), lambda i, k: (0, 0)),
            pl.BlockSpec((1, d_pad), lambda i, k: (0, 0)),
            pl.BlockSpec((d_pad, o_pad), lambda i, k: (0, 0)),
            pl.BlockSpec((d_pad, o_pad), lambda i, k: (0, 0)),
            pl.BlockSpec((1, o_pad), lambda i, k: (0, 0)),
        ],
        out_specs=pl.BlockSpec((tm, o_pad), lambda i, k: (i * nk + k, 0)),
        scratch_shapes=[pltpu.VMEM((n_pad, d_pad), jnp.bfloat16)],
        compiler_params=pltpu.CompilerParams(
            dimension_semantics=("parallel", "arbitrary"),
            vmem_limit_bytes=64 * 1024 * 1024),
    )(feat_p, alpha_p, alpha_p, wqT_p, bq_p, w1T_p, w2T_p, bw_p)

    return out_p[:n, :out_dim]


# speedup vs baseline: 1.1577x; 1.0443x over previous
import functools

import jax
import jax.numpy as jnp
from jax import lax
from jax.experimental import pallas as pl
from jax.experimental.pallas import tpu as pltpu


def _round_up(x, m):
    return ((x + m - 1) // m) * m


def _body(feat_ref, alpha_hbm, wqT_ref, bq_ref, w1T_ref, w2T_ref, bw_ref,
          out_ref, abuf, h_ref, sems, *, nk, tk, depth):
    for s in range(depth):
        pltpu.make_async_copy(
            alpha_hbm.at[pl.ds(s * tk, tk), :],
            abuf.at[pl.ds(s * tk, tk), :],
            sems.at[s]).start()

    x = feat_ref[...].astype(jnp.bfloat16)
    acc = jnp.dot(x, wqT_ref[...], preferred_element_type=jnp.float32)
    h_ref[...] = jnp.maximum(acc + bq_ref[...], 0.0).astype(jnp.bfloat16)

    def step(k, carry):
        slot = lax.rem(k, depth)
        row0 = slot * tk
        pltpu.make_async_copy(
            alpha_hbm.at[pl.ds(k * tk, tk), :],
            abuf.at[pl.ds(row0, tk), :],
            sems.at[slot]).wait()
        a16 = abuf[pl.ds(row0, tk), :].astype(jnp.bfloat16)
        hn = jnp.dot(a16, h_ref[...], preferred_element_type=jnp.float32)

        @pl.when(k + depth < nk)
        def _():
            pltpu.make_async_copy(
                alpha_hbm.at[pl.ds((k + depth) * tk, tk), :],
                abuf.at[pl.ds(row0, tk), :],
                sems.at[slot]).start()

        hd = h_ref[pl.ds(k * tk, tk), :]
        z = jnp.dot(hd, w1T_ref[...], preferred_element_type=jnp.float32)
        z = z + jnp.dot(hn.astype(jnp.bfloat16), w2T_ref[...],
                        preferred_element_type=jnp.float32)
        z = jnp.maximum(z + bw_ref[...], 0.0)
        sumsq = jnp.sum(z * z, axis=-1, keepdims=True)
        inv_norm = lax.rsqrt(sumsq + 1e-12)
        out_ref[pl.ds(k * tk, tk), :] = (z * inv_norm).astype(out_ref.dtype)
        return carry

    lax.fori_loop(0, nk, step, 0)


def kernel(features, alpha, wq, bq, ww, bw):
    n, in_dim = features.shape
    out_dim = ww.shape[0]
    dtype = features.dtype

    d_pad = _round_up(in_dim, 128)
    o_pad = _round_up(out_dim, 128)
    n_pad = _round_up(n, 128)

    def pad2(x, r, c):
        if x.shape == (r, c):
            return x
        return jnp.pad(x, ((0, r - x.shape[0]), (0, c - x.shape[1])))

    feat_p = pad2(features, n_pad, d_pad)
    alpha_p = pad2(alpha, n_pad, n_pad)
    wqT_p = pad2(wq.T, d_pad, d_pad).astype(jnp.bfloat16)
    bq_p = pad2(bq.reshape(1, in_dim), 1, d_pad)
    w1T_p = pad2(ww[:, :in_dim].T, d_pad, o_pad).astype(jnp.bfloat16)
    w2T_p = pad2(ww[:, in_dim:].T, d_pad, o_pad).astype(jnp.bfloat16)
    bw_p = pad2(bw.reshape(1, out_dim), 1, o_pad)

    tk = 512 if n_pad % 512 == 0 else 128
    nk = n_pad // tk
    depth = min(3, nk)

    out_p = pl.pallas_call(
        functools.partial(_body, nk=nk, tk=tk, depth=depth),
        out_shape=jax.ShapeDtypeStruct((n_pad, o_pad), dtype),
        in_specs=[
            pl.BlockSpec((n_pad, d_pad), lambda: (0, 0)),
            pl.BlockSpec(memory_space=pl.ANY),
            pl.BlockSpec((d_pad, d_pad), lambda: (0, 0)),
            pl.BlockSpec((1, d_pad), lambda: (0, 0)),
            pl.BlockSpec((d_pad, o_pad), lambda: (0, 0)),
            pl.BlockSpec((d_pad, o_pad), lambda: (0, 0)),
            pl.BlockSpec((1, o_pad), lambda: (0, 0)),
        ],
        out_specs=pl.BlockSpec((n_pad, o_pad), lambda: (0, 0)),
        scratch_shapes=[
            pltpu.VMEM((depth * tk, n_pad), jnp.float32),
            pltpu.VMEM((n_pad, d_pad), jnp.bfloat16),
            pltpu.SemaphoreType.DMA((depth,)),
        ],
        compiler_params=pltpu.CompilerParams(
            vmem_limit_bytes=58 * 1024 * 1024),
    )(feat_p, alpha_p, wqT_p, bq_p, w1T_p, w2T_p, bw_p)

    return out_p[:n, :out_dim]
